# baseline (speedup 1.0000x reference)
import functools

import jax
import jax.numpy as jnp
from jax import lax
from jax.experimental import pallas as pl
from jax.experimental.pallas import tpu as pltpu
from jax.experimental.pallas import tpu_sc as plsc

N = 10000
E = 320000
HID = 256
NC = 2
NS = 16
K = 128
ROWS_PER_CORE = (2 * E) // K // NC
BATCH_STEPS = -(-ROWS_PER_CORE // NS)
CHUNK = 632
TAIL = N - (NS - 1) * CHUNK


def _for_chunk(s, fn):
    @pl.when(s < NS - 1)
    def _():
        fn(pl.multiple_of(s * CHUNK, 8), CHUNK)

    @pl.when(s == NS - 1)
    def _():
        fn((NS - 1) * CHUNK, TAIL)


GRP = 8


@functools.cache
def _make_seg_sum(rows_per_core):
    mesh = plsc.VectorSubcoreMesh(core_axis_name="c", subcore_axis_name="s",
                                  num_cores=NC, num_subcores=NS)
    bpc = rows_per_core // NS
    n_grp = bpc // GRP

    @functools.partial(
        pl.kernel,
        out_type=[jax.ShapeDtypeStruct((NC, N, 128), jnp.float32)],
        mesh=mesh,
        scratch_types=[
            pltpu.VMEM((GRP, K), jnp.int32),
            pltpu.VMEM((GRP, K), jnp.int32),
            pltpu.VMEM((K, 128), jnp.float32),
            pltpu.VMEM_SHARED((N + 8, 128), jnp.float32),
            pltpu.SemaphoreType.DMA,
        ])
    def body(xflat, src3d, dst3d, z128, out, srcg, dstg, rows, acc, gsem):
        c = lax.axis_index("c")
        s = lax.axis_index("s")
        _for_chunk(s, lambda st, sz: pltpu.sync_copy(
            z128.at[pl.ds(st, sz)], acc.at[pl.ds(st, sz)]))
        plsc.subcore_barrier()

        base = s * bpc

        def group(g, carry):
            r0 = pl.multiple_of(base + g * GRP, GRP)
            pltpu.sync_copy(src3d.at[c, pl.ds(r0, GRP)], srcg)
            pltpu.sync_copy(dst3d.at[c, pl.ds(r0, GRP)], dstg)
            for j in range(GRP):
                pltpu.async_copy(xflat.at[srcg.at[j]], rows, gsem).wait()
                pltpu.sync_copy(rows, acc.at[dstg.at[j]], add=True)
            return carry

        lax.fori_loop(0, n_grp, group, 0)
        plsc.subcore_barrier()
        _for_chunk(s, lambda st, sz: pltpu.sync_copy(
            acc.at[pl.ds(st, sz)], out.at[c, pl.ds(st, sz)]))

    return body


@functools.cache
def _make_count(rows_per_core):
    mesh = plsc.VectorSubcoreMesh(core_axis_name="c", subcore_axis_name="s",
                                  num_cores=NC, num_subcores=NS)
    bpc = rows_per_core // NS
    n_grp = bpc // GRP

    @functools.partial(
        pl.kernel,
        out_type=[jax.ShapeDtypeStruct((NC, N, 128), jnp.float32)],
        mesh=mesh,
        scratch_types=[
            pltpu.VMEM((GRP, K), jnp.int32),
            pltpu.VMEM((K, 128), jnp.float32),
            pltpu.VMEM_SHARED((N + 8, 128), jnp.float32),
        ])
    def body(dst3d, z128, ones, out, dstg, onesb, acc):
        c = lax.axis_index("c")
        s = lax.axis_index("s")
        _for_chunk(s, lambda st, sz: pltpu.sync_copy(
            z128.at[pl.ds(st, sz)], acc.at[pl.ds(st, sz)]))
        pltpu.sync_copy(ones, onesb)
        plsc.subcore_barrier()

        base = s * bpc

        def group(g, carry):
            r0 = pl.multiple_of(base + g * GRP, GRP)
            pltpu.sync_copy(dst3d.at[c, pl.ds(r0, GRP)], dstg)
            for j in range(GRP):
                pltpu.sync_copy(onesb, acc.at[dstg.at[j]], add=True)
            return carry

        lax.fori_loop(0, n_grp, group, 0)
        plsc.subcore_barrier()
        _for_chunk(s, lambda st, sz: pltpu.sync_copy(
            acc.at[pl.ds(st, sz)], out.at[c, pl.ds(st, sz)]))

    return body


def _make_layer_a(din, split, nb=1000):
    dh = din // 2
    nsteps = N // nb

    def body(agg_ref, cnt_ref, h_ref, wl_ref, wr_ref, bl_ref,
             t_ref, s_ref, ss_ref):
        count = cnt_ref[0, :, 0:1] + cnt_ref[1, :, 0:1]
        inv = 1.0 / jnp.maximum(count, 1.0)
        if split:
            t = (jnp.dot(agg_ref[0] * inv, wl_ref[0:dh, :],
                         preferred_element_type=jnp.float32)
                 + jnp.dot(agg_ref[1] * inv, wl_ref[dh:din, :],
                           preferred_element_type=jnp.float32)
                 + jnp.dot(h_ref[0], wr_ref[0:dh, :],
                           preferred_element_type=jnp.float32)
                 + jnp.dot(h_ref[1], wr_ref[dh:din, :],
                           preferred_element_type=jnp.float32))
        else:
            t = (jnp.dot((agg_ref[0] + agg_ref[1]) * inv, wl_ref[...],
                         preferred_element_type=jnp.float32)
                 + jnp.dot(h_ref[...], wr_ref[...],
                           preferred_element_type=jnp.float32))
        t = t + bl_ref[0:1, :]
        t_ref[...] = t

        @pl.when(pl.program_id(0) == 0)
        def _():
            s_ref[...] = jnp.zeros_like(s_ref)
            ss_ref[...] = jnp.zeros_like(ss_ref)

        s_ref[...] += jnp.sum(t, axis=0, keepdims=True)
        ss_ref[...] += jnp.sum(t * t, axis=0, keepdims=True)

    h_spec = (pl.BlockSpec((2, nb, dh), lambda i: (0, i, 0)) if split
              else pl.BlockSpec((nb, din), lambda i: (i, 0)))
    return pl.pallas_call(
        body,
        grid=(nsteps,),
        in_specs=[
            pl.BlockSpec((2, nb, 128), lambda i: (0, i, 0)),
            pl.BlockSpec((2, nb, 128), lambda i: (0, i, 0)),
            h_spec,
            pl.BlockSpec((din, HID), lambda i: (0, 0)),
            pl.BlockSpec((din, HID), lambda i: (0, 0)),
            pl.BlockSpec((1, HID), lambda i: (0, 0)),
        ],
        out_specs=[
            pl.BlockSpec((nb, HID), lambda i: (i, 0)),
            pl.BlockSpec((1, HID), lambda i: (0, 0)),
            pl.BlockSpec((1, HID), lambda i: (0, 0)),
        ],
        out_shape=[
            jax.ShapeDtypeStruct((N, HID), jnp.float32),
            jax.ShapeDtypeStruct((1, HID), jnp.float32),
            jax.ShapeDtypeStruct((1, HID), jnp.float32),
        ],
    )


def _make_layer_b(residual, classify, nb=1000):
    nsteps = N // nb

    def body(t_ref, s_ref, ss_ref, g_ref, be_ref, *rest):
        mu = s_ref[0:1, :] * (1.0 / N)
        var = ss_ref[0:1, :] * (1.0 / N) - mu * mu
        rstd = lax.rsqrt(var + 1e-5)
        h = (t_ref[...] - mu) * rstd * g_ref[0:1, :] + be_ref[0:1, :]
        h = jnp.maximum(h, 0.0)
        if classify:
            wct_ref, bc_ref, out_ref = rest
            out_ref[...] = (jnp.dot(h, wct_ref[...],
                                    preferred_element_type=jnp.float32)
                            + bc_ref[0:1, :])
        else:
            if residual:
                hin_ref, out_ref = rest
            else:
                (out_ref,) = rest
            ha = h[:, 0:HID // 2]
            hb = h[:, HID // 2:HID]
            if residual:
                ha = ha + hin_ref[0]
                hb = hb + hin_ref[1]
            out_ref[0] = ha
            out_ref[1] = hb

    in_specs = [
        pl.BlockSpec((nb, HID), lambda i: (i, 0)),
        pl.BlockSpec((1, HID), lambda i: (0, 0)),
        pl.BlockSpec((1, HID), lambda i: (0, 0)),
        pl.BlockSpec((1, HID), lambda i: (0, 0)),
        pl.BlockSpec((1, HID), lambda i: (0, 0)),
    ]
    if classify:
        in_specs += [
            pl.BlockSpec((HID, 128), lambda i: (0, 0)),
            pl.BlockSpec((1, 128), lambda i: (0, 0)),
        ]
        out_specs = pl.BlockSpec((nb, 128), lambda i: (i, 0))
        out_shape = jax.ShapeDtypeStruct((N, 128), jnp.float32)
    else:
        if residual:
            in_specs.append(pl.BlockSpec((2, nb, HID // 2),
                                         lambda i: (0, i, 0)))
        out_specs = pl.BlockSpec((2, nb, HID // 2), lambda i: (0, i, 0))
        out_shape = jax.ShapeDtypeStruct((2, N, HID // 2), jnp.float32)

    return pl.pallas_call(
        body,
        grid=(nsteps,),
        in_specs=in_specs,
        out_specs=out_specs,
        out_shape=out_shape,
    )


_layer_a_128 = _make_layer_a(128, split=False)
_layer_a_256 = _make_layer_a(256, split=True)
_layer_b_first = _make_layer_b(residual=False, classify=False)
_layer_b_res = _make_layer_b(residual=True, classify=False)
_layer_b_cls = _make_layer_b(residual=False, classify=True)


def kernel(x, edge_index, Wl0, bl0, Wr0, g0, be0, Wl1, bl1, Wr1, g1, be1,
           Wl2, bl2, Wr2, g2, be2, Wc, bc):
    f32 = jnp.float32
    src = edge_index[0].astype(jnp.int32)
    dst = edge_index[1].astype(jnp.int32)
    rpc_e = -(-(E // NC) // (K * NS * GRP)) * NS * GRP
    rpc_c = -(-E // (K * NS * GRP)) * NS * GRP

    def pad_rows(a, rows, val):
        return jnp.concatenate(
            [a, jnp.full((rows * K - a.shape[0],), val, a.dtype)]
        ).reshape(rows, K)

    eh = E // NC
    src3d_e = jnp.stack([pad_rows(src[:eh], rpc_e, 0),
                         pad_rows(src[eh:], rpc_e, 0)])
    dst3d_e = jnp.stack([pad_rows(dst[:eh], rpc_e, N),
                         pad_rows(dst[eh:], rpc_e, N)])
    src_pad = pad_rows(src, rpc_c, 0)
    dst_pad = pad_rows(dst, rpc_c, N)
    src3d_c = jnp.stack([src_pad, src_pad + N])
    dst3d_c = jnp.stack([dst_pad, dst_pad])
    z128 = jnp.zeros((N, 128), f32)
    ones = jnp.ones((K, 128), f32)

    [cnt] = _make_count(rpc_e)(dst3d_e, z128, ones)
    [agg0] = _make_seg_sum(rpc_e)(x, src3d_e, dst3d_e, z128)
    t0, s0, ss0 = _layer_a_128(agg0, cnt, x,
                               Wl0.T, Wr0.T, bl0.reshape(1, HID))
    h1 = _layer_b_first(t0, s0, ss0, g0.reshape(1, HID), be0.reshape(1, HID))

    [agg1] = _make_seg_sum(rpc_c)(h1.reshape(2 * N, 128), src3d_c,
                                  dst3d_c, z128)
    t1, s1, ss1 = _layer_a_256(agg1, cnt, h1,
                               Wl1.T, Wr1.T, bl1.reshape(1, HID))
    h2 = _layer_b_res(t1, s1, ss1, g1.reshape(1, HID), be1.reshape(1, HID),
                      h1)

    [agg2] = _make_seg_sum(rpc_c)(h2.reshape(2 * N, 128), src3d_c,
                                  dst3d_c, z128)
    t2, s2, ss2 = _layer_a_256(agg2, cnt, h2,
                               Wl2.T, Wr2.T, bl2.reshape(1, HID))
    wct = jnp.zeros((HID, 128), f32).at[:, 0].set(Wc[0, :])
    bcp = jnp.zeros((1, 128), f32).at[0, 0].set(bc[0])
    out = _layer_b_cls(t2, s2, ss2, g2.reshape(1, HID), be2.reshape(1, HID),
                       wct, bcp)
    return out[:, 0]


# speedup vs baseline: 1.9743x; 1.9743x over previous
import functools

import jax
import jax.numpy as jnp
from jax import lax
from jax.experimental import pallas as pl
from jax.experimental.pallas import tpu as pltpu
from jax.experimental.pallas import tpu_sc as plsc

N = 10000
E = 320000
HID = 256
NC = 2
NS = 16
K = 128
ROWS_PER_CORE = (2 * E) // K // NC
BATCH_STEPS = -(-ROWS_PER_CORE // NS)
CHUNK = 632
TAIL = N - (NS - 1) * CHUNK


def _for_chunk(s, fn):
    @pl.when(s < NS - 1)
    def _():
        fn(pl.multiple_of(s * CHUNK, 8), CHUNK)

    @pl.when(s == NS - 1)
    def _():
        fn((NS - 1) * CHUNK, TAIL)


GRP = 8


@functools.cache
def _make_seg_sum(rows_per_core, real_rpc):
    mesh = plsc.VectorSubcoreMesh(core_axis_name="c", subcore_axis_name="s",
                                  num_cores=NC, num_subcores=NS)
    bpc = rows_per_core // NS
    n_grp = bpc // GRP

    @functools.partial(
        pl.kernel,
        out_type=[jax.ShapeDtypeStruct((NC, N, 128), jnp.float32)],
        mesh=mesh,
        scratch_types=[
            pltpu.VMEM((GRP, K), jnp.int32),
            pltpu.VMEM((GRP, K), jnp.int32),
            pltpu.VMEM((K, 128), jnp.float32),
            pltpu.VMEM_SHARED((N, 128), jnp.float32),
            pltpu.SemaphoreType.DMA,
        ])
    def body(xflat, src3d, dst3d, z128, out, srcg, dstg, rows, acc, gsem):
        c = lax.axis_index("c")
        s = lax.axis_index("s")
        _for_chunk(s, lambda st, sz: pltpu.sync_copy(
            z128.at[pl.ds(st, sz)], acc.at[pl.ds(st, sz)]))
        plsc.subcore_barrier()

        base = s * bpc

        def group(g, carry):
            r0 = pl.multiple_of(base + g * GRP, GRP)
            pltpu.sync_copy(src3d.at[c, pl.ds(r0, GRP)], srcg)
            pltpu.sync_copy(dst3d.at[c, pl.ds(r0, GRP)], dstg)
            for j in range(GRP):
                @pl.when(r0 + j < real_rpc)
                def _():
                    pltpu.async_copy(xflat.at[srcg.at[j]], rows,
                                     gsem).wait()
                    pltpu.sync_copy(rows, acc.at[dstg.at[j]], add=True)
            return carry

        lax.fori_loop(0, n_grp, group, 0)
        plsc.subcore_barrier()
        _for_chunk(s, lambda st, sz: pltpu.sync_copy(
            acc.at[pl.ds(st, sz)], out.at[c, pl.ds(st, sz)]))

    return body


@functools.cache
def _make_count(rows_per_core, real_rpc):
    mesh = plsc.VectorSubcoreMesh(core_axis_name="c", subcore_axis_name="s",
                                  num_cores=NC, num_subcores=NS)
    bpc = rows_per_core // NS
    n_grp = bpc // GRP

    @functools.partial(
        pl.kernel,
        out_type=[jax.ShapeDtypeStruct((NC, N, 128), jnp.float32)],
        mesh=mesh,
        scratch_types=[
            pltpu.VMEM((GRP, K), jnp.int32),
            pltpu.VMEM((K, 128), jnp.float32),
            pltpu.VMEM_SHARED((N, 128), jnp.float32),
        ])
    def body(dst3d, z128, ones, out, dstg, onesb, acc):
        c = lax.axis_index("c")
        s = lax.axis_index("s")
        _for_chunk(s, lambda st, sz: pltpu.sync_copy(
            z128.at[pl.ds(st, sz)], acc.at[pl.ds(st, sz)]))
        pltpu.sync_copy(ones, onesb)
        plsc.subcore_barrier()

        base = s * bpc

        def group(g, carry):
            r0 = pl.multiple_of(base + g * GRP, GRP)
            pltpu.sync_copy(dst3d.at[c, pl.ds(r0, GRP)], dstg)
            for j in range(GRP):
                @pl.when(r0 + j < real_rpc)
                def _():
                    pltpu.sync_copy(onesb, acc.at[dstg.at[j]], add=True)
            return carry

        lax.fori_loop(0, n_grp, group, 0)
        plsc.subcore_barrier()
        _for_chunk(s, lambda st, sz: pltpu.sync_copy(
            acc.at[pl.ds(st, sz)], out.at[c, pl.ds(st, sz)]))

    return body


def _make_layer_a(din, split, nb=1000):
    dh = din // 2
    nsteps = N // nb

    def body(agg_ref, cnt_ref, h_ref, wl_ref, wr_ref, bl_ref,
             t_ref, s_ref, ss_ref):
        count = cnt_ref[0, :, 0:1] + cnt_ref[1, :, 0:1]
        inv = 1.0 / jnp.maximum(count, 1.0)
        if split:
            t = (jnp.dot(agg_ref[0] * inv, wl_ref[0:dh, :],
                         preferred_element_type=jnp.float32)
                 + jnp.dot(agg_ref[1] * inv, wl_ref[dh:din, :],
                           preferred_element_type=jnp.float32)
                 + jnp.dot(h_ref[0], wr_ref[0:dh, :],
                           preferred_element_type=jnp.float32)
                 + jnp.dot(h_ref[1], wr_ref[dh:din, :],
                           preferred_element_type=jnp.float32))
        else:
            t = (jnp.dot((agg_ref[0] + agg_ref[1]) * inv, wl_ref[...],
                         preferred_element_type=jnp.float32)
                 + jnp.dot(h_ref[...], wr_ref[...],
                           preferred_element_type=jnp.float32))
        t = t + bl_ref[0:1, :]
        t_ref[...] = t

        @pl.when(pl.program_id(0) == 0)
        def _():
            s_ref[...] = jnp.zeros_like(s_ref)
            ss_ref[...] = jnp.zeros_like(ss_ref)

        s_ref[...] += jnp.sum(t, axis=0, keepdims=True)
        ss_ref[...] += jnp.sum(t * t, axis=0, keepdims=True)

    h_spec = (pl.BlockSpec((2, nb, dh), lambda i: (0, i, 0)) if split
              else pl.BlockSpec((nb, din), lambda i: (i, 0)))
    return pl.pallas_call(
        body,
        grid=(nsteps,),
        in_specs=[
            pl.BlockSpec((2, nb, 128), lambda i: (0, i, 0)),
            pl.BlockSpec((2, nb, 128), lambda i: (0, i, 0)),
            h_spec,
            pl.BlockSpec((din, HID), lambda i: (0, 0)),
            pl.BlockSpec((din, HID), lambda i: (0, 0)),
            pl.BlockSpec((1, HID), lambda i: (0, 0)),
        ],
        out_specs=[
            pl.BlockSpec((nb, HID), lambda i: (i, 0)),
            pl.BlockSpec((1, HID), lambda i: (0, 0)),
            pl.BlockSpec((1, HID), lambda i: (0, 0)),
        ],
        out_shape=[
            jax.ShapeDtypeStruct((N, HID), jnp.float32),
            jax.ShapeDtypeStruct((1, HID), jnp.float32),
            jax.ShapeDtypeStruct((1, HID), jnp.float32),
        ],
    )


def _make_layer_b(residual, classify, nb=1000):
    nsteps = N // nb

    def body(t_ref, s_ref, ss_ref, g_ref, be_ref, *rest):
        mu = s_ref[0:1, :] * (1.0 / N)
        var = ss_ref[0:1, :] * (1.0 / N) - mu * mu
        rstd = lax.rsqrt(var + 1e-5)
        h = (t_ref[...] - mu) * rstd * g_ref[0:1, :] + be_ref[0:1, :]
        h = jnp.maximum(h, 0.0)
        if classify:
            wct_ref, bc_ref, out_ref = rest
            out_ref[...] = (jnp.dot(h, wct_ref[...],
                                    preferred_element_type=jnp.float32)
                            + bc_ref[0:1, :])
        else:
            if residual:
                hin_ref, out_ref = rest
            else:
                (out_ref,) = rest
            ha = h[:, 0:HID // 2]
            hb = h[:, HID // 2:HID]
            if residual:
                ha = ha + hin_ref[0]
                hb = hb + hin_ref[1]
            out_ref[0] = ha
            out_ref[1] = hb

    in_specs = [
        pl.BlockSpec((nb, HID), lambda i: (i, 0)),
        pl.BlockSpec((1, HID), lambda i: (0, 0)),
        pl.BlockSpec((1, HID), lambda i: (0, 0)),
        pl.BlockSpec((1, HID), lambda i: (0, 0)),
        pl.BlockSpec((1, HID), lambda i: (0, 0)),
    ]
    if classify:
        in_specs += [
            pl.BlockSpec((HID, 128), lambda i: (0, 0)),
            pl.BlockSpec((1, 128), lambda i: (0, 0)),
        ]
        out_specs = pl.BlockSpec((nb, 128), lambda i: (i, 0))
        out_shape = jax.ShapeDtypeStruct((N, 128), jnp.float32)
    else:
        if residual:
            in_specs.append(pl.BlockSpec((2, nb, HID // 2),
                                         lambda i: (0, i, 0)))
        out_specs = pl.BlockSpec((2, nb, HID // 2), lambda i: (0, i, 0))
        out_shape = jax.ShapeDtypeStruct((2, N, HID // 2), jnp.float32)

    return pl.pallas_call(
        body,
        grid=(nsteps,),
        in_specs=in_specs,
        out_specs=out_specs,
        out_shape=out_shape,
    )


_layer_a_128 = _make_layer_a(128, split=False)
_layer_a_256 = _make_layer_a(256, split=True)
_layer_b_first = _make_layer_b(residual=False, classify=False)
_layer_b_res = _make_layer_b(residual=True, classify=False)
_layer_b_cls = _make_layer_b(residual=False, classify=True)


def kernel(x, edge_index, Wl0, bl0, Wr0, g0, be0, Wl1, bl1, Wr1, g1, be1,
           Wl2, bl2, Wr2, g2, be2, Wc, bc):
    f32 = jnp.float32
    src = edge_index[0].astype(jnp.int32)
    dst = edge_index[1].astype(jnp.int32)
    rpc_e = -(-(E // NC) // (K * NS * GRP)) * NS * GRP
    rpc_c = -(-E // (K * NS * GRP)) * NS * GRP

    def pad_rows(a, rows, val):
        return jnp.concatenate(
            [a, jnp.full((rows * K - a.shape[0],), val, a.dtype)]
        ).reshape(rows, K)

    eh = E // NC
    src3d_e = jnp.stack([pad_rows(src[:eh], rpc_e, 0),
                         pad_rows(src[eh:], rpc_e, 0)])
    dst3d_e = jnp.stack([pad_rows(dst[:eh], rpc_e, N),
                         pad_rows(dst[eh:], rpc_e, N)])
    src_pad = pad_rows(src, rpc_c, 0)
    dst_pad = pad_rows(dst, rpc_c, N)
    src3d_c = jnp.stack([src_pad, src_pad + N])
    dst3d_c = jnp.stack([dst_pad, dst_pad])
    z128 = jnp.zeros((N, 128), f32)
    ones = jnp.ones((K, 128), f32)

    real_e = E // NC // K
    real_c = E // K
    [cnt] = _make_count(rpc_e, real_e)(dst3d_e, z128, ones)
    [agg0] = _make_seg_sum(rpc_e, real_e)(x, src3d_e, dst3d_e, z128)
    t0, s0, ss0 = _layer_a_128(agg0, cnt, x,
                               Wl0.T, Wr0.T, bl0.reshape(1, HID))
    h1 = _layer_b_first(t0, s0, ss0, g0.reshape(1, HID), be0.reshape(1, HID))

    [agg1] = _make_seg_sum(rpc_c, real_c)(h1.reshape(2 * N, 128), src3d_c,
                                          dst3d_c, z128)
    t1, s1, ss1 = _layer_a_256(agg1, cnt, h1,
                               Wl1.T, Wr1.T, bl1.reshape(1, HID))
    h2 = _layer_b_res(t1, s1, ss1, g1.reshape(1, HID), be1.reshape(1, HID),
                      h1)

    [agg2] = _make_seg_sum(rpc_c, real_c)(h2.reshape(2 * N, 128), src3d_c,
                                          dst3d_c, z128)
    t2, s2, ss2 = _layer_a_256(agg2, cnt, h2,
                               Wl2.T, Wr2.T, bl2.reshape(1, HID))
    wct = jnp.zeros((HID, 128), f32).at[:, 0].set(Wc[0, :])
    bcp = jnp.zeros((1, 128), f32).at[0, 0].set(bc[0])
    out = _layer_b_cls(t2, s2, ss2, g2.reshape(1, HID), be2.reshape(1, HID),
                       wct, bcp)
    return out[:, 0]
